# Initial kernel scaffold; baseline (speedup 1.0000x reference)
import functools

import jax
import jax.numpy as jnp
from jax import lax
from jax.experimental import pallas as pl
from jax.experimental.pallas import tpu as pltpu
from jax.experimental.pallas import tpu_sc as plsc

_N = 10000
_EH = 5000
_NNZ = 320000
_H = 128
_C = 40
_NP = 10240
_EHP = 5120
_CHUNK = 128
_NCHUNKS = _NNZ // _CHUNK
_NC = 2
_NS = 16

_MESH = plsc.VectorSubcoreMesh(
    core_axis_name="c", subcore_axis_name="s", num_cores=_NC, num_subcores=_NS)

_F32 = jnp.float32
_I32 = jnp.int32


def _zero_1d(ref, n):
  def body(j, _):
    ref[pl.ds(j * 16, 16)] = jnp.zeros((16,), _F32)
    return 0
  lax.fori_loop(0, n // 16, body, 0)


def _deg_body(ni_hbm, ei_hbm, dv_out, de_out,
              dv_s, de_s, zbuf, ones_v, ni_v, ei_v):
  c = lax.axis_index("c")
  s = lax.axis_index("s")
  w = s * _NC + c

  _zero_1d(zbuf, 640)
  def ones_body(j, _):
    ones_v[pl.ds(j * 16, 16)] = jnp.ones((16,), _F32)
    return 0
  lax.fori_loop(0, _CHUNK // 16, ones_body, 0)

  pltpu.sync_copy(zbuf, dv_s.at[pl.ds(s * 640, 640)])
  pltpu.sync_copy(zbuf.at[pl.ds(0, 320)], de_s.at[pl.ds(s * 320, 320)])
  plsc.subcore_barrier()

  trips = 78 + jnp.where(w < 4, 1, 0)
  def body(it, _):
    off = (it * 32 + w) * _CHUNK
    pltpu.sync_copy(ni_hbm.at[pl.ds(off, _CHUNK)], ni_v)
    pltpu.sync_copy(ei_hbm.at[pl.ds(off, _CHUNK)], ei_v)
    pltpu.sync_copy(ones_v, dv_s.at[ni_v], add=True)
    pltpu.sync_copy(ones_v, de_s.at[ei_v], add=True)
    return 0
  lax.fori_loop(0, trips, body, 0)

  plsc.subcore_barrier()
  pltpu.sync_copy(dv_s.at[pl.ds(s * 640, 640)], dv_out.at[c, pl.ds(s * 640, 640)])
  pltpu.sync_copy(de_s.at[pl.ds(s * 320, 320)], de_out.at[c, pl.ds(s * 320, 320)])


_deg_kernel = pl.kernel(
    _deg_body,
    out_type=(jax.ShapeDtypeStruct((_NC, _NP), _F32),
              jax.ShapeDtypeStruct((_NC, _EHP), _F32)),
    mesh=_MESH,
    scratch_types=[
        pltpu.VMEM_SHARED((_NP,), _F32),
        pltpu.VMEM_SHARED((_EHP,), _F32),
        pltpu.VMEM((640,), _F32),
        pltpu.VMEM((_CHUNK,), _F32),
        pltpu.VMEM((_CHUNK,), _I32),
        pltpu.VMEM((_CHUNK,), _I32),
    ],
)


def _norm_body(ni_hbm, ei_hbm, wv_hbm, we_hbm, ne_out, nv_out,
               ne_s, nv_s, wv_v, we_v, zbuf, ni_v, ei_v, vn_v, ve_v):
  c = lax.axis_index("c")
  s = lax.axis_index("s")
  w = s * _NC + c

  _zero_1d(zbuf, 640)
  pltpu.sync_copy(zbuf, nv_s.at[pl.ds(s * 640, 640)])
  pltpu.sync_copy(zbuf.at[pl.ds(0, 320)], ne_s.at[pl.ds(s * 320, 320)])
  pltpu.sync_copy(wv_hbm, wv_v)
  pltpu.sync_copy(we_hbm, we_v)
  plsc.subcore_barrier()

  trips = 78 + jnp.where(w < 4, 1, 0)
  def body(it, _):
    off = (it * 32 + w) * _CHUNK
    pltpu.sync_copy(ni_hbm.at[pl.ds(off, _CHUNK)], ni_v)
    pltpu.sync_copy(ei_hbm.at[pl.ds(off, _CHUNK)], ei_v)
    for g in range(_CHUNK // 16):
      idx_n = ni_v[pl.ds(g * 16, 16)]
      ve_v[pl.ds(g * 16, 16)] = plsc.load_gather(wv_v, [idx_n])
      idx_e = ei_v[pl.ds(g * 16, 16)]
      vn_v[pl.ds(g * 16, 16)] = plsc.load_gather(we_v, [idx_e])
    pltpu.sync_copy(ve_v, ne_s.at[ei_v], add=True)
    pltpu.sync_copy(vn_v, nv_s.at[ni_v], add=True)
    return 0
  lax.fori_loop(0, trips, body, 0)

  plsc.subcore_barrier()
  pltpu.sync_copy(ne_s.at[pl.ds(s * 320, 320)], ne_out.at[c, pl.ds(s * 320, 320)])
  pltpu.sync_copy(nv_s.at[pl.ds(s * 640, 640)], nv_out.at[c, pl.ds(s * 640, 640)])


_norm_kernel = pl.kernel(
    _norm_body,
    out_type=(jax.ShapeDtypeStruct((_NC, _EHP), _F32),
              jax.ShapeDtypeStruct((_NC, _NP), _F32)),
    mesh=_MESH,
    scratch_types=[
        pltpu.VMEM_SHARED((_EHP,), _F32),
        pltpu.VMEM_SHARED((_NP,), _F32),
        pltpu.VMEM((_NP,), _F32),
        pltpu.VMEM((_EHP,), _F32),
        pltpu.VMEM((640,), _F32),
        pltpu.VMEM((_CHUNK,), _I32),
        pltpu.VMEM((_CHUNK,), _I32),
        pltpu.VMEM((_CHUNK,), _F32),
        pltpu.VMEM((_CHUNK,), _F32),
    ],
)


def _make_spmm(rows_p):
  rpt = rows_p // _NS
  half = _NCHUNKS // _NC
  base_trips = half // _NS
  rem = half - base_trips * _NS

  def body(z_hbm, gi_hbm, si_hbm, out_hbm,
           acc_s, zbuf, gi_v, si_v, rows_v, sem):
    c = lax.axis_index("c")
    s = lax.axis_index("s")

    def zrow(i, _):
      for j in range(_H // 16):
        zbuf[i, pl.ds(j * 16, 16)] = jnp.zeros((16,), _F32)
      return 0
    lax.fori_loop(0, 40, zrow, 0)
    for k in range(rpt // 40):
      pltpu.sync_copy(zbuf, acc_s.at[pl.ds(s * rpt + k * 40, 40)])
    plsc.subcore_barrier()

    trips = base_trips + jnp.where(s < rem, 1, 0)
    def loop(it, _):
      off = (c * half + it * _NS + s) * _CHUNK
      pltpu.sync_copy(gi_hbm.at[pl.ds(off, _CHUNK)], gi_v)
      pltpu.sync_copy(si_hbm.at[pl.ds(off, _CHUNK)], si_v)
      pltpu.async_copy(z_hbm.at[gi_v], rows_v, sem).wait()
      pltpu.sync_copy(rows_v, acc_s.at[si_v], add=True)
      return 0
    lax.fori_loop(0, trips, loop, 0)

    plsc.subcore_barrier()
    for k in range(rpt // 320):
      r = s * rpt + k * 320
      pltpu.sync_copy(acc_s.at[pl.ds(r, 320)], out_hbm.at[c, pl.ds(r, 320)])

  return pl.kernel(
      body,
      out_type=jax.ShapeDtypeStruct((_NC, rows_p, _H), _F32),
      mesh=_MESH,
      scratch_types=[
          pltpu.VMEM_SHARED((rows_p, _H), _F32),
          pltpu.VMEM((40, _H), _F32),
          pltpu.VMEM((_CHUNK,), _I32),
          pltpu.VMEM((_CHUNK,), _I32),
          pltpu.VMEM((_CHUNK, _H), _F32),
          pltpu.SemaphoreType.DMA,
      ],
  )


_spmm_edge = _make_spmm(_EHP)
_spmm_node = _make_spmm(_NP)


def _pow_body(dv_ref, de_ref, wv_ref, we_ref):
  dv = jnp.maximum(dv_ref[0] + dv_ref[1], 1.0)
  rv = lax.rsqrt(dv)
  wv_ref[...] = rv * rv * rv
  de = jnp.maximum(de_ref[0] + de_ref[1], 1.0)
  we_ref[...] = lax.rsqrt(de)


_pow_kernel = pl.pallas_call(
    _pow_body,
    out_shape=(jax.ShapeDtypeStruct((_NP // 128, 128), _F32),
               jax.ShapeDtypeStruct((_EHP // 128, 128), _F32)),
)


def _m0_body(x_ref, wi_ref, bi_ref, we_ref, be_ref, wv_ref, z_ref):
  t = jnp.dot(x_ref[...], wi_ref[...], preferred_element_type=_F32) + bi_ref[...]
  z = jnp.dot(t, we_ref[...], preferred_element_type=_F32) + be_ref[...]
  z_ref[...] = z * wv_ref[...]


def _mid_body(p0_ref, p1_ref, n0_ref, n1_ref, wm_ref, bm_ref, ws_ref, z_ref):
  x = jnp.maximum(
      (p0_ref[...] + p1_ref[...])
      / jnp.maximum(n0_ref[...] + n1_ref[...], 1e-12), 0.0)
  z = jnp.dot(x, wm_ref[...], preferred_element_type=_F32) + bm_ref[...]
  z_ref[...] = z * ws_ref[...]


def _mid_keep_body(p0_ref, p1_ref, n0_ref, n1_ref, wm_ref, bm_ref, ws_ref,
                   x_ref, z_ref):
  x = jnp.maximum(
      (p0_ref[...] + p1_ref[...])
      / jnp.maximum(n0_ref[...] + n1_ref[...], 1e-12), 0.0)
  x_ref[...] = x
  z = jnp.dot(x, wm_ref[...], preferred_element_type=_F32) + bm_ref[...]
  z_ref[...] = z * ws_ref[...]


def _out_body(p0_ref, p1_ref, n0_ref, n1_ref, wo_ref, bo_ref, y_ref):
  x = jnp.maximum(
      (p0_ref[...] + p1_ref[...])
      / jnp.maximum(n0_ref[...] + n1_ref[...], 1e-12), 0.0)
  y_ref[...] = jnp.dot(x, wo_ref[...], preferred_element_type=_F32) + bo_ref[...]


def _row_spec(blk):
  return pl.BlockSpec((blk, _H), lambda i: (i, 0))


def _col_spec(blk):
  return pl.BlockSpec((blk, 1), lambda i: (i, 0))


_W_SPEC = pl.BlockSpec((_H, _H), lambda i: (0, 0))
_B_SPEC = pl.BlockSpec((1, _H), lambda i: (0, 0))

_m0_kernel = pl.pallas_call(
    _m0_body,
    grid=(25,),
    in_specs=[_row_spec(400), _W_SPEC, _B_SPEC, _W_SPEC, _B_SPEC, _col_spec(400)],
    out_specs=_row_spec(400),
    out_shape=jax.ShapeDtypeStruct((_N, _H), _F32),
)


def _make_mid(rows, blk, keep_x):
  body = _mid_keep_body if keep_x else _mid_body
  out_spec = _row_spec(blk)
  out_shape = jax.ShapeDtypeStruct((rows, _H), _F32)
  return pl.pallas_call(
      body,
      grid=(rows // blk,),
      in_specs=[_row_spec(blk), _row_spec(blk), _col_spec(blk), _col_spec(blk),
                _W_SPEC, _B_SPEC, _col_spec(blk)],
      out_specs=(out_spec, out_spec) if keep_x else out_spec,
      out_shape=(out_shape, out_shape) if keep_x else out_shape,
  )


_m_edge = _make_mid(_EH, 200, keep_x=False)
_m_edge_keep = _make_mid(_EH, 200, keep_x=True)
_m_node = _make_mid(_N, 400, keep_x=False)

_m_out = pl.pallas_call(
    _out_body,
    grid=(25,),
    in_specs=[_row_spec(400), _row_spec(400), _col_spec(400), _col_spec(400),
              _W_SPEC, _B_SPEC],
    out_specs=_row_spec(400),
    out_shape=jax.ShapeDtypeStruct((_N, _H), _F32),
)


@jax.jit
def kernel(x_0, node_idx, edge_idx, W_in, b_in, W0_e, b0_e, W0_v, b0_v,
           W1_e, b1_e, W1_v, b1_v, W_out, b_out):
  ni = node_idx.astype(_I32)
  ei = edge_idx.astype(_I32)

  dv_p, de_p = _deg_kernel(ni, ei)
  wv2d, we2d = _pow_kernel(dv_p.reshape(_NC, _NP // 128, 128),
                           de_p.reshape(_NC, _EHP // 128, 128))
  wv = wv2d.reshape(_NP)
  we = we2d.reshape(_EHP)
  ne_p, nv_p = _norm_kernel(ni, ei, wv, we)

  wv_col = wv[:_N, None]
  we_col = we[:_EH, None]
  ne0 = ne_p[0, :_EH, None]
  ne1 = ne_p[1, :_EH, None]
  nv0 = nv_p[0, :_N, None]
  nv1 = nv_p[1, :_N, None]

  b_in2 = b_in.reshape(1, _H)
  b0e2 = b0_e.reshape(1, _H)
  b0v2 = b0_v.reshape(1, _H)
  b1e2 = b1_e.reshape(1, _H)
  b1v2 = b1_v.reshape(1, _H)
  wo_pad = jnp.pad(W_out, ((0, 0), (0, _H - _C)))
  bo_pad = jnp.pad(b_out, (0, _H - _C)).reshape(1, _H)

  z0 = _m0_kernel(x_0, W_in, b_in2, W0_e, b0e2, wv_col)
  pe = _spmm_edge(z0, ni, ei)
  z1 = _m_edge(pe[0, :_EH], pe[1, :_EH], ne0, ne1, W0_v, b0v2, we_col)
  pv = _spmm_node(z1, ei, ni)
  z2 = _m_node(pv[0, :_N], pv[1, :_N], nv0, nv1, W1_e, b1e2, wv_col)
  pe2 = _spmm_edge(z2, ni, ei)
  x1, z3 = _m_edge_keep(pe2[0, :_EH], pe2[1, :_EH], ne0, ne1, W1_v, b1v2, we_col)
  pv2 = _spmm_node(z3, ei, ni)
  y_pad = _m_out(pv2[0, :_N], pv2[1, :_N], nv0, nv1, wo_pad, bo_pad)
  return (y_pad[:, :_C], x1)


# speedup vs baseline: 7.6663x; 7.6663x over previous
import functools

import jax
import jax.numpy as jnp
from jax import lax
from jax.experimental import pallas as pl
from jax.experimental.pallas import tpu as pltpu
from jax.experimental.pallas import tpu_sc as plsc

_N = 10000
_EH = 5000
_NNZ = 320000
_H = 128
_C = 40
_NP = 10240
_EHP = 5120
_CHUNK = 128
_NCHUNKS = _NNZ // _CHUNK
_NC = 2
_NS = 16

_MESH = plsc.VectorSubcoreMesh(
    core_axis_name="c", subcore_axis_name="s", num_cores=_NC, num_subcores=_NS)

_F32 = jnp.float32
_I32 = jnp.int32


def _zero_1d(ref, n):
  def body(j, _):
    ref[pl.ds(j * 16, 16)] = jnp.zeros((16,), _F32)
    return 0
  lax.fori_loop(0, n // 16, body, 0)


def _deg_body(ni_hbm, ei_hbm, dv_out, de_out,
              dv_s, de_s, zbuf, ones_v, ni_v, ei_v):
  c = lax.axis_index("c")
  s = lax.axis_index("s")
  w = s * _NC + c

  _zero_1d(zbuf, 640)
  def ones_body(j, _):
    ones_v[pl.ds(j * 16, 16)] = jnp.ones((16,), _F32)
    return 0
  lax.fori_loop(0, _CHUNK // 16, ones_body, 0)

  pltpu.sync_copy(zbuf, dv_s.at[pl.ds(s * 640, 640)])
  pltpu.sync_copy(zbuf.at[pl.ds(0, 320)], de_s.at[pl.ds(s * 320, 320)])
  plsc.subcore_barrier()

  trips = 78 + jnp.where(w < 4, 1, 0)
  def body(it, _):
    off = (it * 32 + w) * _CHUNK
    pltpu.sync_copy(ni_hbm.at[pl.ds(off, _CHUNK)], ni_v)
    pltpu.sync_copy(ei_hbm.at[pl.ds(off, _CHUNK)], ei_v)
    pltpu.sync_copy(ones_v, dv_s.at[ni_v], add=True)
    pltpu.sync_copy(ones_v, de_s.at[ei_v], add=True)
    return 0
  lax.fori_loop(0, trips, body, 0)

  plsc.subcore_barrier()
  pltpu.sync_copy(dv_s.at[pl.ds(s * 640, 640)], zbuf)
  pltpu.sync_copy(zbuf, dv_out.at[pl.ds(c * _NP + s * 640, 640)])
  pltpu.sync_copy(de_s.at[pl.ds(s * 320, 320)], zbuf.at[pl.ds(0, 320)])
  pltpu.sync_copy(zbuf.at[pl.ds(0, 320)],
                  de_out.at[pl.ds(c * _EHP + s * 320, 320)])


_deg_kernel = pl.kernel(
    _deg_body,
    out_type=(jax.ShapeDtypeStruct((_NC * _NP,), _F32),
              jax.ShapeDtypeStruct((_NC * _EHP,), _F32)),
    mesh=_MESH,
    scratch_types=[
        pltpu.VMEM_SHARED((_NP,), _F32),
        pltpu.VMEM_SHARED((_EHP,), _F32),
        pltpu.VMEM((640,), _F32),
        pltpu.VMEM((_CHUNK,), _F32),
        pltpu.VMEM((_CHUNK,), _I32),
        pltpu.VMEM((_CHUNK,), _I32),
    ],
)


def _norm_body(ni_hbm, ei_hbm, wv_hbm, we_hbm, ne_out, nv_out,
               ne_s, nv_s, zbuf, ni_v, ei_v, vn_v, ve_v, sem):
  c = lax.axis_index("c")
  s = lax.axis_index("s")
  w = s * _NC + c

  _zero_1d(zbuf, 640)
  pltpu.sync_copy(zbuf, nv_s.at[pl.ds(s * 640, 640)])
  pltpu.sync_copy(zbuf.at[pl.ds(0, 320)], ne_s.at[pl.ds(s * 320, 320)])
  plsc.subcore_barrier()

  trips = 78 + jnp.where(w < 4, 1, 0)
  def body(it, _):
    off = (it * 32 + w) * _CHUNK
    pltpu.sync_copy(ni_hbm.at[pl.ds(off, _CHUNK)], ni_v)
    pltpu.sync_copy(ei_hbm.at[pl.ds(off, _CHUNK)], ei_v)
    pltpu.async_copy(wv_hbm.at[ni_v], ve_v, sem).wait()
    pltpu.async_copy(we_hbm.at[ei_v], vn_v, sem).wait()
    pltpu.sync_copy(ve_v, ne_s.at[ei_v], add=True)
    pltpu.sync_copy(vn_v, nv_s.at[ni_v], add=True)
    return 0
  lax.fori_loop(0, trips, body, 0)

  plsc.subcore_barrier()
  pltpu.sync_copy(ne_s.at[pl.ds(s * 320, 320)], zbuf.at[pl.ds(0, 320)])
  pltpu.sync_copy(zbuf.at[pl.ds(0, 320)],
                  ne_out.at[pl.ds(c * _EHP + s * 320, 320)])
  pltpu.sync_copy(nv_s.at[pl.ds(s * 640, 640)], zbuf)
  pltpu.sync_copy(zbuf, nv_out.at[pl.ds(c * _NP + s * 640, 640)])


_norm_kernel = pl.kernel(
    _norm_body,
    out_type=(jax.ShapeDtypeStruct((_NC * _EHP,), _F32),
              jax.ShapeDtypeStruct((_NC * _NP,), _F32)),
    mesh=_MESH,
    scratch_types=[
        pltpu.VMEM_SHARED((_EHP,), _F32),
        pltpu.VMEM_SHARED((_NP,), _F32),
        pltpu.VMEM((640,), _F32),
        pltpu.VMEM((_CHUNK,), _I32),
        pltpu.VMEM((_CHUNK,), _I32),
        pltpu.VMEM((_CHUNK,), _F32),
        pltpu.VMEM((_CHUNK,), _F32),
        pltpu.SemaphoreType.DMA,
    ],
)


def _make_spmm(rows_p):
  rpt = rows_p // _NS
  half = _NCHUNKS // _NC
  base_trips = half // _NS
  rem = half - base_trips * _NS

  def body(z_hbm, gi_hbm, si_hbm, out_hbm,
           acc_s, zbuf, gi_v, si_v, rows_v, sem):
    c = lax.axis_index("c")
    s = lax.axis_index("s")

    def zrow(i, _):
      for j in range(_H // 16):
        zbuf[i, pl.ds(j * 16, 16)] = jnp.zeros((16,), _F32)
      return 0
    lax.fori_loop(0, 40, zrow, 0)
    for k in range(rpt // 40):
      pltpu.sync_copy(zbuf, acc_s.at[pl.ds(s * rpt + k * 40, 40)])
    plsc.subcore_barrier()

    trips = base_trips + jnp.where(s < rem, 1, 0)
    def loop(it, _):
      off = (c * half + it * _NS + s) * _CHUNK
      pltpu.sync_copy(gi_hbm.at[pl.ds(off, _CHUNK)], gi_v)
      pltpu.sync_copy(si_hbm.at[pl.ds(off, _CHUNK)], si_v)
      pltpu.async_copy(z_hbm.at[gi_v], rows_v, sem).wait()
      pltpu.sync_copy(rows_v, acc_s.at[si_v], add=True)
      return 0
    lax.fori_loop(0, trips, loop, 0)

    plsc.subcore_barrier()
    for k in range(rpt // 320):
      r = s * rpt + k * 320
      pltpu.sync_copy(acc_s.at[pl.ds(r, 320)], out_hbm.at[c, pl.ds(r, 320)])

  return pl.kernel(
      body,
      out_type=jax.ShapeDtypeStruct((_NC, rows_p, _H), _F32),
      mesh=_MESH,
      scratch_types=[
          pltpu.VMEM_SHARED((rows_p, _H), _F32),
          pltpu.VMEM((40, _H), _F32),
          pltpu.VMEM((_CHUNK,), _I32),
          pltpu.VMEM((_CHUNK,), _I32),
          pltpu.VMEM((_CHUNK, _H), _F32),
          pltpu.SemaphoreType.DMA,
      ],
  )


_spmm_edge = _make_spmm(_EHP)
_spmm_node = _make_spmm(_NP)


def _pow_body(dv_ref, de_ref, wv_ref, we_ref):
  dv = jnp.maximum(dv_ref[0] + dv_ref[1], 1.0)
  rv = lax.rsqrt(dv)
  wv_ref[...] = rv * rv * rv
  de = jnp.maximum(de_ref[0] + de_ref[1], 1.0)
  we_ref[...] = lax.rsqrt(de)


_pow_kernel = pl.pallas_call(
    _pow_body,
    out_shape=(jax.ShapeDtypeStruct((_NP // 128, 128), _F32),
               jax.ShapeDtypeStruct((_EHP // 128, 128), _F32)),
)


def _m0_body(x_ref, wi_ref, bi_ref, we_ref, be_ref, wv_ref, z_ref):
  t = jnp.dot(x_ref[...], wi_ref[...], preferred_element_type=_F32) + bi_ref[...]
  z = jnp.dot(t, we_ref[...], preferred_element_type=_F32) + be_ref[...]
  z_ref[...] = z * wv_ref[...]


def _mid_body(p0_ref, p1_ref, n0_ref, n1_ref, wm_ref, bm_ref, ws_ref, z_ref):
  x = jnp.maximum(
      (p0_ref[...] + p1_ref[...])
      / jnp.maximum(n0_ref[...] + n1_ref[...], 1e-12), 0.0)
  z = jnp.dot(x, wm_ref[...], preferred_element_type=_F32) + bm_ref[...]
  z_ref[...] = z * ws_ref[...]


def _mid_keep_body(p0_ref, p1_ref, n0_ref, n1_ref, wm_ref, bm_ref, ws_ref,
                   x_ref, z_ref):
  x = jnp.maximum(
      (p0_ref[...] + p1_ref[...])
      / jnp.maximum(n0_ref[...] + n1_ref[...], 1e-12), 0.0)
  x_ref[...] = x
  z = jnp.dot(x, wm_ref[...], preferred_element_type=_F32) + bm_ref[...]
  z_ref[...] = z * ws_ref[...]


def _out_body(p0_ref, p1_ref, n0_ref, n1_ref, wo_ref, bo_ref, y_ref):
  x = jnp.maximum(
      (p0_ref[...] + p1_ref[...])
      / jnp.maximum(n0_ref[...] + n1_ref[...], 1e-12), 0.0)
  y_ref[...] = jnp.dot(x, wo_ref[...], preferred_element_type=_F32) + bo_ref[...]


def _row_spec(blk):
  return pl.BlockSpec((blk, _H), lambda i: (i, 0))


def _col_spec(blk):
  return pl.BlockSpec((blk, 1), lambda i: (i, 0))


_W_SPEC = pl.BlockSpec((_H, _H), lambda i: (0, 0))
_B_SPEC = pl.BlockSpec((1, _H), lambda i: (0, 0))

_m0_kernel = pl.pallas_call(
    _m0_body,
    grid=(25,),
    in_specs=[_row_spec(400), _W_SPEC, _B_SPEC, _W_SPEC, _B_SPEC, _col_spec(400)],
    out_specs=_row_spec(400),
    out_shape=jax.ShapeDtypeStruct((_N, _H), _F32),
)


def _make_mid(rows, blk, keep_x):
  body = _mid_keep_body if keep_x else _mid_body
  out_spec = _row_spec(blk)
  out_shape = jax.ShapeDtypeStruct((rows, _H), _F32)
  return pl.pallas_call(
      body,
      grid=(rows // blk,),
      in_specs=[_row_spec(blk), _row_spec(blk), _col_spec(blk), _col_spec(blk),
                _W_SPEC, _B_SPEC, _col_spec(blk)],
      out_specs=(out_spec, out_spec) if keep_x else out_spec,
      out_shape=(out_shape, out_shape) if keep_x else out_shape,
  )


_m_edge = _make_mid(_EH, 200, keep_x=False)
_m_edge_keep = _make_mid(_EH, 200, keep_x=True)
_m_node = _make_mid(_N, 400, keep_x=False)

_m_out = pl.pallas_call(
    _out_body,
    grid=(25,),
    in_specs=[_row_spec(400), _row_spec(400), _col_spec(400), _col_spec(400),
              _W_SPEC, _B_SPEC],
    out_specs=_row_spec(400),
    out_shape=jax.ShapeDtypeStruct((_N, _H), _F32),
)


@jax.jit
def kernel(x_0, node_idx, edge_idx, W_in, b_in, W0_e, b0_e, W0_v, b0_v,
           W1_e, b1_e, W1_v, b1_v, W_out, b_out):
  ni = node_idx.astype(_I32)
  ei = edge_idx.astype(_I32)

  dv_p, de_p = _deg_kernel(ni, ei)
  wv2d, we2d = _pow_kernel(dv_p.reshape(_NC, _NP // 128, 128),
                           de_p.reshape(_NC, _EHP // 128, 128))
  wv = wv2d.reshape(_NP)
  we = we2d.reshape(_EHP)
  ne_p, nv_p = _norm_kernel(ni, ei, wv, we)
  ne_p = ne_p.reshape(_NC, _EHP)
  nv_p = nv_p.reshape(_NC, _NP)

  wv_col = wv[:_N, None]
  we_col = we[:_EH, None]
  ne0 = ne_p[0, :_EH, None]
  ne1 = ne_p[1, :_EH, None]
  nv0 = nv_p[0, :_N, None]
  nv1 = nv_p[1, :_N, None]

  b_in2 = b_in.reshape(1, _H)
  b0e2 = b0_e.reshape(1, _H)
  b0v2 = b0_v.reshape(1, _H)
  b1e2 = b1_e.reshape(1, _H)
  b1v2 = b1_v.reshape(1, _H)
  wo_pad = jnp.pad(W_out, ((0, 0), (0, _H - _C)))
  bo_pad = jnp.pad(b_out, (0, _H - _C)).reshape(1, _H)

  z0 = _m0_kernel(x_0, W_in, b_in2, W0_e, b0e2, wv_col)
  pe = _spmm_edge(z0, ni, ei)
  z1 = _m_edge(pe[0, :_EH], pe[1, :_EH], ne0, ne1, W0_v, b0v2, we_col)
  pv = _spmm_node(z1, ei, ni)
  z2 = _m_node(pv[0, :_N], pv[1, :_N], nv0, nv1, W1_e, b1e2, wv_col)
  pe2 = _spmm_edge(z2, ni, ei)
  x1, z3 = _m_edge_keep(pe2[0, :_EH], pe2[1, :_EH], ne0, ne1, W1_v, b1v2, we_col)
  pv2 = _spmm_node(z3, ei, ni)
  y_pad = _m_out(pv2[0, :_N], pv2[1, :_N], nv0, nv1, wo_pad, bo_pad)
  return (y_pad[:, :_C], x1)
